# Initial kernel scaffold; baseline (speedup 1.0000x reference)
import jax
import jax.numpy as jnp
from jax.experimental import pallas as pl


def kernel(x, edge_index, edge_attr, W1, b1, W2, b2, eps_node):
    raise NotImplementedError("write your pallas kernel here")


# speedup vs baseline: 4.0302x; 4.0302x over previous
import functools

import jax
import jax.numpy as jnp
from jax import lax
from jax.experimental import pallas as pl
from jax.experimental.pallas import tpu as pltpu
from jax.experimental.pallas import tpu_sc as plsc

N_NODES = 10000
N_EDGES = 320000
D = 128

NC = 2
NS = 16
NW = NC * NS

CH = 128
TOTAL_CHUNKS = N_EDGES // CH
CHUNKS_PER_TILE = -(-TOTAL_CHUNKS // NW)

ROW_CH = 80
ROW_CHUNKS = N_NODES // ROW_CH


def _sc_message_scatter(x, src, dst, edge_attr):
    mesh = plsc.VectorSubcoreMesh(
        core_axis_name="c", subcore_axis_name="s", num_cores=NC, num_subcores=NS
    )

    @functools.partial(
        pl.kernel,
        out_type=jax.ShapeDtypeStruct((NC, N_NODES, D), jnp.float32),
        mesh=mesh,
        scratch_types=[
            pltpu.VMEM((CH, D), jnp.float32),
            pltpu.VMEM((CH, D), jnp.float32),
            pltpu.VMEM((CH,), jnp.int32),
            pltpu.VMEM((CH,), jnp.int32),
            pltpu.VMEM_SHARED((N_NODES, D), jnp.float32),
            pltpu.SemaphoreType.DMA,
        ],
    )
    def body(x_hbm, src_hbm, dst_hbm, attr_hbm, out_hbm,
             attr_buf, xrows_buf, src_buf, dst_buf, acc, sem):
        c = lax.axis_index("c")
        s = lax.axis_index("s")
        w = s * NC + c

        def zrow(r, _):
            for k in range(D // 16):
                attr_buf[r, pl.ds(k * 16, 16)] = jnp.zeros((16,), jnp.float32)
            return 0
        lax.fori_loop(0, ROW_CH, zrow, 0)
        n_stripes = (ROW_CHUNKS - s + NS - 1) // NS

        def zstripe(t, _):
            base = (s + NS * t) * ROW_CH
            pltpu.sync_copy(attr_buf.at[pl.ds(0, ROW_CH)],
                            acc.at[pl.ds(base, ROW_CH)])
            return 0
        lax.fori_loop(0, n_stripes, zstripe, 0)
        plsc.subcore_barrier()

        lo = w * CHUNKS_PER_TILE
        hi = jnp.minimum(lo + CHUNKS_PER_TILE, TOTAL_CHUNKS)

        def chunk(i, _):
            base = i * CH
            pltpu.sync_copy(src_hbm.at[pl.ds(base, CH)], src_buf)
            pltpu.sync_copy(dst_hbm.at[pl.ds(base, CH)], dst_buf)
            pltpu.sync_copy(attr_hbm.at[pl.ds(base, CH)], attr_buf)
            pltpu.async_copy(x_hbm.at[src_buf], xrows_buf, sem).wait()

            def edge(e, _):
                for k in range(D // 16):
                    sl = pl.ds(k * 16, 16)
                    attr_buf[e, sl] = jnp.maximum(
                        attr_buf[e, sl] + xrows_buf[e, sl], 0.0)
                return 0
            lax.fori_loop(0, CH, edge, 0)

            pltpu.sync_copy(attr_buf, acc.at[dst_buf], add=True)
            return 0
        lax.fori_loop(lo, hi, chunk, 0)

        plsc.subcore_barrier()

        def wstripe(t, _):
            base = (s + NS * t) * ROW_CH
            pltpu.sync_copy(acc.at[pl.ds(base, ROW_CH)],
                            out_hbm.at[c, pl.ds(base, ROW_CH)])
            return 0
        lax.fori_loop(0, n_stripes, wstripe, 0)

    return body(x, src, dst, edge_attr)


ROWS_PER_BLOCK = 2000


def _mlp_body(x_ref, p0_ref, p1_ref, eps_ref, w1_ref, b1_ref, w2_ref, b2_ref,
              o_ref):
    h = (1.0 + eps_ref[0, 0]) * x_ref[...] + p0_ref[...] + p1_ref[...]
    h = jnp.maximum(
        jnp.dot(h, w1_ref[...], preferred_element_type=jnp.float32)
        + b1_ref[...], 0.0)
    o_ref[...] = (jnp.dot(h, w2_ref[...], preferred_element_type=jnp.float32)
                  + b2_ref[...])


def _tc_mlp(x, p0, p1, eps_node, W1, b1, W2, b2):
    grid = (N_NODES // ROWS_PER_BLOCK,)
    row_spec = pl.BlockSpec((ROWS_PER_BLOCK, D), lambda i: (i, 0))
    mat_spec = pl.BlockSpec((D, D), lambda i: (0, 0))
    vec_spec = pl.BlockSpec((1, D), lambda i: (0, 0))
    return pl.pallas_call(
        _mlp_body,
        grid=grid,
        in_specs=[
            row_spec, row_spec, row_spec,
            pl.BlockSpec(memory_space=pltpu.SMEM),
            mat_spec, vec_spec, mat_spec, vec_spec,
        ],
        out_specs=row_spec,
        out_shape=jax.ShapeDtypeStruct((N_NODES, D), jnp.float32),
    )(x, p0, p1, eps_node, W1, b1.reshape(1, D), W2, b2.reshape(1, D))


def kernel(x, edge_index, edge_attr, W1, b1, W2, b2, eps_node):
    src = edge_index[0].astype(jnp.int32)
    dst = edge_index[1].astype(jnp.int32)
    partial = _sc_message_scatter(x, src, dst, edge_attr)
    return _tc_mlp(x, partial[0], partial[1], eps_node, W1, b1, W2, b2)


# speedup vs baseline: 8.6151x; 2.1377x over previous
import functools

import jax
import jax.numpy as jnp
from jax import lax
from jax.experimental import pallas as pl
from jax.experimental.pallas import tpu as pltpu
from jax.experimental.pallas import tpu_sc as plsc

N_NODES = 10000
N_EDGES = 320000
D = 128
NVR = D // 16

NC = 2
NS = 16
NW = NC * NS

CH = 80
TOTAL_CHUNKS = N_EDGES // CH
CPT = TOTAL_CHUNKS // NW

ROW_CH = 80
ROW_CHUNKS = N_NODES // ROW_CH


def _sc_message_scatter(x, src, dst, edge_attr):
    mesh = plsc.VectorSubcoreMesh(
        core_axis_name="c", subcore_axis_name="s", num_cores=NC, num_subcores=NS
    )

    @functools.partial(
        pl.kernel,
        out_type=jax.ShapeDtypeStruct((NC, N_NODES, D), jnp.float32),
        mesh=mesh,
        scratch_types=[
            pltpu.VMEM((CH, D), jnp.float32),
            pltpu.VMEM((CH, D), jnp.float32),
            pltpu.VMEM((CH, D), jnp.float32),
            pltpu.VMEM((CH, D), jnp.float32),
            pltpu.VMEM((CH,), jnp.int32),
            pltpu.VMEM((CH,), jnp.int32),
            pltpu.VMEM((CH,), jnp.int32),
            pltpu.VMEM((CH,), jnp.int32),
            pltpu.VMEM_SHARED((N_NODES, D), jnp.float32),
            pltpu.SemaphoreType.DMA,
            pltpu.SemaphoreType.DMA,
            pltpu.SemaphoreType.DMA,
            pltpu.SemaphoreType.DMA,
            pltpu.SemaphoreType.DMA,
            pltpu.SemaphoreType.DMA,
            pltpu.SemaphoreType.DMA,
            pltpu.SemaphoreType.DMA,
            pltpu.SemaphoreType.DMA,
            pltpu.SemaphoreType.DMA,
        ],
    )
    def body(x_hbm, src_hbm, dst_hbm, attr_hbm, out_hbm,
             attr_a, attr_b, xr_a, xr_b, si_a, si_b, di_a, di_b, acc,
             sem_aa, sem_ab, sem_ga, sem_gb, sem_sa, sem_sb,
             sem_ia, sem_ib, sem_da, sem_db):
        c = lax.axis_index("c")
        s = lax.axis_index("s")
        w = s * NC + c

        def zrow(r, _):
            for k in range(NVR):
                attr_a[r, pl.ds(k * 16, 16)] = jnp.zeros((16,), jnp.float32)
            return 0
        lax.fori_loop(0, ROW_CH, zrow, 0)
        n_stripes = (ROW_CHUNKS - s + NS - 1) // NS

        def zstripe(t, _):
            base = (s + NS * t) * ROW_CH
            pltpu.sync_copy(attr_a, acc.at[pl.ds(base, ROW_CH)])
            return 0
        lax.fori_loop(0, n_stripes, zstripe, 0)
        plsc.subcore_barrier()

        c0 = w * CPT

        def start_src(t, si, sem):
            pltpu.async_copy(src_hbm.at[pl.ds((c0 + t) * CH, CH)], si, sem)

        def start_dst(t, di, sem):
            pltpu.async_copy(dst_hbm.at[pl.ds((c0 + t) * CH, CH)], di, sem)

        def start_attr(t, attr_buf, sem):
            pltpu.async_copy(attr_hbm.at[pl.ds((c0 + t) * CH, CH)], attr_buf,
                             sem)

        def wait_sem(hbm_ref, ref, sem):
            pltpu.make_async_copy(hbm_ref.at[pl.ds(0, CH)], ref, sem).wait()

        def start_gather(si, xr_buf, sem):
            pltpu.async_copy(x_hbm.at[si], xr_buf, sem)

        def wait_gather(si, xr_buf, sem):
            pltpu.make_async_copy(x_hbm.at[si], xr_buf, sem).wait()

        def compute(attr_buf, xr_buf):
            @plsc.parallel_loop(0, CH, step=1, unroll=2)
            def edge(e):
                for k in range(NVR):
                    sl = pl.ds(k * 16, 16)
                    attr_buf[e, sl] = jnp.maximum(
                        attr_buf[e, sl] + xr_buf[e, sl], 0.0)

        def start_scatter(di, attr_buf, sem):
            pltpu.async_copy(attr_buf, acc.at[di], sem, add=True)

        def wait_scatter(di, attr_buf, sem):
            pltpu.make_async_copy(attr_buf, acc.at[di], sem).wait()

        start_src(0, si_a, sem_ia)
        start_dst(0, di_a, sem_da)
        start_attr(0, attr_a, sem_aa)
        start_src(1, si_b, sem_ib)
        start_dst(1, di_b, sem_db)
        start_attr(1, attr_b, sem_ab)
        wait_sem(src_hbm, si_a, sem_ia)
        start_gather(si_a, xr_a, sem_ga)
        wait_sem(src_hbm, si_b, sem_ib)
        start_gather(si_b, xr_b, sem_gb)

        def process(t, attr_buf, xr_buf, si, di,
                    sem_a, sem_g, sem_s, sem_i, sem_d, prefetch):
            wait_sem(attr_hbm, attr_buf, sem_a)
            wait_gather(si, xr_buf, sem_g)
            if prefetch:
                start_src(t + 2, si, sem_i)
            compute(attr_buf, xr_buf)
            wait_sem(dst_hbm, di, sem_d)
            start_scatter(di, attr_buf, sem_s)
            if prefetch:
                wait_sem(src_hbm, si, sem_i)
                start_gather(si, xr_buf, sem_g)
            wait_scatter(di, attr_buf, sem_s)
            if prefetch:
                start_dst(t + 2, di, sem_d)
                start_attr(t + 2, attr_buf, sem_a)

        def pbody(j, _):
            a = 2 * j
            process(a, attr_a, xr_a, si_a, di_a,
                    sem_aa, sem_ga, sem_sa, sem_ia, sem_da, True)
            process(a + 1, attr_b, xr_b, si_b, di_b,
                    sem_ab, sem_gb, sem_sb, sem_ib, sem_db, True)
            return 0
        lax.fori_loop(0, (CPT - 3) // 2, pbody, 0)
        t = CPT - 3
        wait_sem(attr_hbm, attr_a, sem_aa)
        wait_gather(si_a, xr_a, sem_ga)
        start_src(t + 2, si_a, sem_ia)
        compute(attr_a, xr_a)
        wait_sem(dst_hbm, di_a, sem_da)
        start_scatter(di_a, attr_a, sem_sa)
        wait_sem(src_hbm, si_a, sem_ia)
        start_gather(si_a, xr_a, sem_ga)
        wait_scatter(di_a, attr_a, sem_sa)
        start_dst(t + 2, di_a, sem_da)
        start_attr(t + 2, attr_a, sem_aa)
        process(t + 1, attr_b, xr_b, si_b, di_b,
                sem_ab, sem_gb, sem_sb, sem_ib, sem_db, False)
        process(t + 2, attr_a, xr_a, si_a, di_a,
                sem_aa, sem_ga, sem_sa, sem_ia, sem_da, False)

        plsc.subcore_barrier()

        def wstripe(t, _):
            base = (s + NS * t) * ROW_CH
            pltpu.sync_copy(acc.at[pl.ds(base, ROW_CH)],
                            out_hbm.at[c, pl.ds(base, ROW_CH)])
            return 0
        lax.fori_loop(0, n_stripes, wstripe, 0)

    return body(x, src, dst, edge_attr)


ROWS_PER_BLOCK = 2000


def _mlp_body(x_ref, p0_ref, p1_ref, eps_ref, w1_ref, b1_ref, w2_ref, b2_ref,
              o_ref):
    h = (1.0 + eps_ref[0, 0]) * x_ref[...] + p0_ref[...] + p1_ref[...]
    h = jnp.maximum(
        jnp.dot(h, w1_ref[...], preferred_element_type=jnp.float32)
        + b1_ref[...], 0.0)
    o_ref[...] = (jnp.dot(h, w2_ref[...], preferred_element_type=jnp.float32)
                  + b2_ref[...])


def _tc_mlp(x, p0, p1, eps_node, W1, b1, W2, b2):
    grid = (N_NODES // ROWS_PER_BLOCK,)
    row_spec = pl.BlockSpec((ROWS_PER_BLOCK, D), lambda i: (i, 0))
    mat_spec = pl.BlockSpec((D, D), lambda i: (0, 0))
    vec_spec = pl.BlockSpec((1, D), lambda i: (0, 0))
    return pl.pallas_call(
        _mlp_body,
        grid=grid,
        in_specs=[
            row_spec, row_spec, row_spec,
            pl.BlockSpec(memory_space=pltpu.SMEM),
            mat_spec, vec_spec, mat_spec, vec_spec,
        ],
        out_specs=row_spec,
        out_shape=jax.ShapeDtypeStruct((N_NODES, D), jnp.float32),
    )(x, p0, p1, eps_node, W1, b1.reshape(1, D), W2, b2.reshape(1, D))


def kernel(x, edge_index, edge_attr, W1, b1, W2, b2, eps_node):
    src = edge_index[0].astype(jnp.int32)
    dst = edge_index[1].astype(jnp.int32)
    partial = _sc_message_scatter(x, src, dst, edge_attr)
    return _tc_mlp(x, partial[0], partial[1], eps_node, W1, b1, W2, b2)
